# baseline (speedup 1.0000x reference)
import functools

import jax
import jax.numpy as jnp
from jax import lax
from jax.experimental import pallas as pl
from jax.experimental.pallas import tpu as pltpu
from jax.experimental.pallas import tpu_sc as plsc

N = 50000
NPAD = 51200
E = 1600000
IN_DIM = 128
HID_DIM = 256
OUT_DIM = 47
PAD_DIM = 64
HALF = 32
K = 10

BN = 1000
BC = 800
EB = 160
NT = 16
NC = 2
NCHUNK = NPAD // NT
SUB = 128
NSUB = NCHUNK // SUB
ET = E // NT
NB = ET // EB
L = 16


def _mlp_body(x_ref, w1_ref, b1_ref, w2_ref, b2_ref, w3_ref, b3_ref, o_ref):
    x = x_ref[...]
    h = jnp.maximum(jnp.dot(x, w1_ref[...], preferred_element_type=jnp.float32)
                    + b1_ref[...][None, :], 0.0)
    h = jnp.maximum(jnp.dot(h, w2_ref[...], preferred_element_type=jnp.float32)
                    + b2_ref[...][None, :], 0.0)
    o_ref[...] = (jnp.dot(h, w3_ref[...], preferred_element_type=jnp.float32)
                  + b3_ref[...][None, :])


def _mlp(feats, W1, b1, W2, b2, W3p, b3p):
    return pl.pallas_call(
        _mlp_body,
        grid=(N // BN,),
        in_specs=[
            pl.BlockSpec((BN, IN_DIM), lambda i: (i, 0)),
            pl.BlockSpec((IN_DIM, HID_DIM), lambda i: (0, 0)),
            pl.BlockSpec((HID_DIM,), lambda i: (0,)),
            pl.BlockSpec((HID_DIM, HID_DIM), lambda i: (0, 0)),
            pl.BlockSpec((HID_DIM,), lambda i: (0,)),
            pl.BlockSpec((HID_DIM, PAD_DIM), lambda i: (0, 0)),
            pl.BlockSpec((PAD_DIM,), lambda i: (0,)),
        ],
        out_specs=pl.BlockSpec((BN, PAD_DIM), lambda i: (i, 0)),
        out_shape=jax.ShapeDtypeStruct((N, PAD_DIM), jnp.float32),
        compiler_params=pltpu.CompilerParams(
            dimension_semantics=("parallel",)),
    )(feats, W1, b1, W2, b2, W3p, b3p)


def _zero16(ref, base):
    ref[pl.ds(base, L)] = jnp.zeros((L,), jnp.float32)


def _splat(ref, idx):
    return plsc.load_gather(ref, [jnp.full((L,), idx, jnp.int32)])


def _sc_body(src_hbm, dst_hbm, h_hbm,
             s_out, deg_out, pA, pB,
             agg, degS,
             sbuf0, sbuf1, dbuf0, dbuf1, rbuf0, rbuf1, abuf, ivd, dch, ones,
             sem_s0, sem_s1, sem_d0, sem_d1, sem_g0, sem_g1, sem_a0, sem_a1):
    cid = lax.axis_index("c")
    sid = lax.axis_index("s")
    nbase = sid * NCHUNK
    ebase = sid * ET
    coff = cid * NPAD
    sbufs = (sbuf0, sbuf1)
    dbufs = (dbuf0, dbuf1)
    rbufs = (rbuf0, rbuf1)
    sem_s = (sem_s0, sem_s1)
    sem_d = (sem_d0, sem_d1)
    sem_g = (sem_g0, sem_g1)
    sem_a = (sem_a0, sem_a1)

    def i_start(i, b):
        e0 = ebase + i * EB
        da = pltpu.async_copy(src_hbm.at[pl.ds(e0, EB)], sbufs[b], sem_s[b])
        db = pltpu.async_copy(dst_hbm.at[pl.ds(e0, EB)], dbufs[b], sem_d[b])
        return da, db

    def addcoff(b):
        @pl.loop(0, EB // L, unroll=4)
        def _(q):
            sbufs[b][pl.ds(q * L, L)] = sbufs[b][pl.ds(q * L, L)] + coff

    def g_start(b, tab):
        return pltpu.async_copy(tab.at[sbufs[b]], rbufs[b], sem_g[b])

    def a_start(b):
        return pltpu.async_copy(rbufs[b], agg.at[dbufs[b]], sem_a[b],
                                add=True)

    def zero_rbuf(b):
        @pl.loop(0, EB * 2, unroll=4)
        def _(i):
            rbufs[b][i // 2, pl.ds((i % 2) * L, L)] = (
                jnp.zeros((L,), jnp.float32))

    @pl.loop(0, NCHUNK // L, unroll=4)
    def _(i):
        _zero16(ivd, i * L)

    @pl.loop(0, EB // L, unroll=4)
    def _(i):
        ones[pl.ds(i * L, L)] = jnp.ones((L,), jnp.float32)

    pltpu.sync_copy(ivd, degS.at[pl.ds(nbase, NCHUNK)])
    plsc.subcore_barrier()

    @pl.loop(0, NB)
    def _(i):
        pltpu.sync_copy(dst_hbm.at[pl.ds(ebase + i * EB, EB)], dbuf0)
        pltpu.sync_copy(ones, degS.at[dbuf0], add=True)

    plsc.subcore_barrier()

    pltpu.sync_copy(degS.at[pl.ds(nbase, NCHUNK)], dch)

    @pl.when(cid == 0)
    def _():
        pltpu.sync_copy(dch, deg_out.at[pl.ds(nbase, NCHUNK)])

    @pl.loop(0, NCHUNK // L, unroll=2)
    def _(i):
        v = dch[pl.ds(i * L, L)]
        bits = plsc.bitcast(v, jnp.int32)
        y = plsc.bitcast(0x5F3759DF - (bits >> 1), jnp.float32)
        y = y * (1.5 - 0.5 * v * y * y)
        y = y * (1.5 - 0.5 * v * y * y)
        y = y * (1.5 - 0.5 * v * y * y)
        ivd[pl.ds(i * L, L)] = 1.0 / v
        dch[pl.ds(i * L, L)] = y

    zero_rbuf(0)

    @pl.loop(0, NSUB)
    def _(j):
        rb = nbase + j * SUB
        pltpu.sync_copy(h_hbm.at[pl.ds(coff + rb, SUB)], abuf)

        @pl.loop(0, SUB, unroll=2)
        def _(r):
            sc = _splat(dch, j * SUB + r)
            abuf[r, pl.ds(0, L)] = abuf[r, pl.ds(0, L)] * sc
            abuf[r, pl.ds(L, L)] = abuf[r, pl.ds(L, L)] * sc

        pltpu.sync_copy(abuf, pA.at[pl.ds(coff + rb, SUB)])
        pltpu.sync_copy(rbuf0.at[pl.ds(0, SUB), :], agg.at[pl.ds(rb, SUB)])

    plsc.subcore_barrier()

    def scatter_pass(tab):
        @pl.loop(0, NB // 2)
        def _(kk):
            i0 = 2 * kk
            s0a, s0b = i_start(i0, 0)
            s1a, s1b = i_start(i0 + 1, 1)
            s0a.wait()
            addcoff(0)
            g0 = g_start(0, tab)
            s1a.wait()
            addcoff(1)
            s0b.wait()
            s1b.wait()
            g0.wait()
            a0 = a_start(0)
            g1 = g_start(1, tab)
            g1.wait()
            a1 = a_start(1)
            a0.wait()
            a1.wait()

        s0a, s0b = i_start(NB - 1, 0)
        s0a.wait()
        addcoff(0)
        g0 = g_start(0, tab)
        s0b.wait()
        g0.wait()
        a0 = a_start(0)
        a0.wait()

        plsc.subcore_barrier()

    def rescale_pass(k, tab_w):
        zero_rbuf(0)

        @pl.loop(0, NSUB)
        def _(j):
            rb = nbase + j * SUB
            pltpu.sync_copy(agg.at[pl.ds(rb, SUB)], abuf)
            pltpu.sync_copy(rbuf0.at[pl.ds(0, SUB), :], agg.at[pl.ds(rb, SUB)])
            pltpu.sync_copy(abuf, s_out.at[k].at[pl.ds(coff + rb, SUB)])

            @pl.loop(0, SUB, unroll=2)
            def _(r):
                sc = _splat(ivd, j * SUB + r)
                abuf[r, pl.ds(0, L)] = abuf[r, pl.ds(0, L)] * sc
                abuf[r, pl.ds(L, L)] = abuf[r, pl.ds(L, L)] * sc

            pltpu.sync_copy(abuf, tab_w.at[pl.ds(coff + rb, SUB)])

        plsc.subcore_barrier()

    @pl.loop(0, K // 2)
    def _(kk):
        scatter_pass(pA)
        rescale_pass(2 * kk, pB)
        scatter_pass(pB)
        rescale_pass(2 * kk + 1, pA)


def _sc_hops(src, dst, h2):
    mesh = plsc.VectorSubcoreMesh(core_axis_name="c", subcore_axis_name="s",
                                  num_cores=NC, num_subcores=NT)
    f = functools.partial(
        pl.kernel,
        out_type=[
            jax.ShapeDtypeStruct((K, NC * NPAD, HALF), jnp.float32),
            jax.ShapeDtypeStruct((NPAD,), jnp.float32),
            jax.ShapeDtypeStruct((NC * NPAD, HALF), jnp.float32),
            jax.ShapeDtypeStruct((NC * NPAD, HALF), jnp.float32),
        ],
        mesh=mesh,
        scratch_types=[
            pltpu.MemorySpace.VMEM_SHARED((NPAD, HALF), jnp.float32),
            pltpu.MemorySpace.VMEM_SHARED((NPAD,), jnp.float32),
            pltpu.VMEM((EB,), jnp.int32),
            pltpu.VMEM((EB,), jnp.int32),
            pltpu.VMEM((EB,), jnp.int32),
            pltpu.VMEM((EB,), jnp.int32),
            pltpu.VMEM((EB, HALF), jnp.float32),
            pltpu.VMEM((EB, HALF), jnp.float32),
            pltpu.VMEM((SUB, HALF), jnp.float32),
            pltpu.VMEM((NCHUNK,), jnp.float32),
            pltpu.VMEM((NCHUNK,), jnp.float32),
            pltpu.VMEM((EB,), jnp.float32),
            pltpu.SemaphoreType.DMA,
            pltpu.SemaphoreType.DMA,
            pltpu.SemaphoreType.DMA,
            pltpu.SemaphoreType.DMA,
            pltpu.SemaphoreType.DMA,
            pltpu.SemaphoreType.DMA,
            pltpu.SemaphoreType.DMA,
            pltpu.SemaphoreType.DMA,
        ],
        compiler_params=pltpu.CompilerParams(needs_layout_passes=False,
                                             use_tc_tiling_on_sc=False),
    )(_sc_body)
    return f(src, dst, h2)


def _combine_body(deg_ref, h_ref, s_ref, sv_ref, o_ref):
    norm = lax.rsqrt(deg_ref[...])
    sv = sv_ref[...]
    f0 = h_ref[...]
    w0 = jax.nn.sigmoid(jnp.sum(f0 * sv, axis=1, keepdims=True))
    acc = w0 * f0
    for k in range(K):
        fk = jnp.concatenate([s_ref[k, 0], s_ref[k, 1]], axis=1) * norm
        wk = jax.nn.sigmoid(jnp.sum(fk * sv, axis=1, keepdims=True))
        acc = acc + wk * fk
    o_ref[...] = acc


def _combine(deg, h, s_all, svp):
    return pl.pallas_call(
        _combine_body,
        grid=(NPAD // BC,),
        in_specs=[
            pl.BlockSpec((BC, 1), lambda i: (i, 0)),
            pl.BlockSpec((BC, PAD_DIM), lambda i: (i, 0)),
            pl.BlockSpec((K, 2, BC, HALF), lambda i: (0, 0, i, 0)),
            pl.BlockSpec((1, PAD_DIM), lambda i: (0, 0)),
        ],
        out_specs=pl.BlockSpec((BC, PAD_DIM), lambda i: (i, 0)),
        out_shape=jax.ShapeDtypeStruct((NPAD, PAD_DIM), jnp.float32),
        compiler_params=pltpu.CompilerParams(
            dimension_semantics=("parallel",)),
    )(deg, h, s_all, svp)


def kernel(feats, edge_index, W1, b1, W2, b2, W3, b3, s):
    srcp = jnp.pad(edge_index[0], (0, 2 * EB))
    dstp = jnp.pad(edge_index[1], (0, 2 * EB))
    W3p = jnp.pad(W3, ((0, 0), (0, PAD_DIM - OUT_DIM)))
    b3p = jnp.pad(b3, (0, PAD_DIM - OUT_DIM))
    svp = jnp.pad(s[:, 0], (0, PAD_DIM - OUT_DIM))[None, :]

    h = _mlp(feats, W1, b1, W2, b2, W3p, b3p)
    hp = jnp.pad(h, ((0, NPAD - N), (0, 0)))
    h2 = jnp.concatenate([hp[:, :HALF], hp[:, HALF:]], axis=0)

    s_out, deg, _pa, _pb = _sc_hops(srcp, dstp, h2)
    s_all = s_out.reshape(K, 2, NPAD, HALF)

    out = _combine(deg[:, None], hp, s_all, svp)
    return out[:N, :OUT_DIM]


# speedup vs baseline: 1.5949x; 1.5949x over previous
import functools

import jax
import jax.numpy as jnp
from jax import lax
from jax.experimental import pallas as pl
from jax.experimental.pallas import tpu as pltpu
from jax.experimental.pallas import tpu_sc as plsc

N = 50000
NPAD = 51200
E = 1600000
IN_DIM = 128
HID_DIM = 256
OUT_DIM = 47
PAD_DIM = 64
HALF = 32
K = 10

BN = 1000
BC = 800
EB = 400
NT = 16
NC = 2
NCHUNK = NPAD // NT
SUB = 128
NSUB = NCHUNK // SUB
ET = E // NT
NB = ET // EB
L = 16


def _mlp_body(x_ref, w1_ref, b1_ref, w2_ref, b2_ref, w3_ref, b3_ref, o_ref):
    x = x_ref[...]
    h = jnp.maximum(jnp.dot(x, w1_ref[...], preferred_element_type=jnp.float32)
                    + b1_ref[...][None, :], 0.0)
    h = jnp.maximum(jnp.dot(h, w2_ref[...], preferred_element_type=jnp.float32)
                    + b2_ref[...][None, :], 0.0)
    o_ref[...] = (jnp.dot(h, w3_ref[...], preferred_element_type=jnp.float32)
                  + b3_ref[...][None, :])


def _mlp(feats, W1, b1, W2, b2, W3p, b3p):
    return pl.pallas_call(
        _mlp_body,
        grid=(N // BN,),
        in_specs=[
            pl.BlockSpec((BN, IN_DIM), lambda i: (i, 0)),
            pl.BlockSpec((IN_DIM, HID_DIM), lambda i: (0, 0)),
            pl.BlockSpec((HID_DIM,), lambda i: (0,)),
            pl.BlockSpec((HID_DIM, HID_DIM), lambda i: (0, 0)),
            pl.BlockSpec((HID_DIM,), lambda i: (0,)),
            pl.BlockSpec((HID_DIM, PAD_DIM), lambda i: (0, 0)),
            pl.BlockSpec((PAD_DIM,), lambda i: (0,)),
        ],
        out_specs=pl.BlockSpec((BN, PAD_DIM), lambda i: (i, 0)),
        out_shape=jax.ShapeDtypeStruct((N, PAD_DIM), jnp.float32),
        compiler_params=pltpu.CompilerParams(
            dimension_semantics=("parallel",)),
    )(feats, W1, b1, W2, b2, W3p, b3p)


def _zero16(ref, base):
    ref[pl.ds(base, L)] = jnp.zeros((L,), jnp.float32)


def _splat(ref, idx):
    return plsc.load_gather(ref, [jnp.full((L,), idx, jnp.int32)])


def _sc_body(src_hbm, dst_hbm, h_hbm,
             s_out, deg_out, pA, pB,
             agg, degS,
             sbuf, sbuf1, dbuf, dbuf1, rbuf, abuf, ivd, dch, ones,
             sem, sem_s0, sem_s1, sem_d0, sem_d1):
    cid = lax.axis_index("c")
    sid = lax.axis_index("s")
    nbase = sid * NCHUNK
    ebase = sid * ET
    coff = cid * NPAD
    sbufs = (sbuf, sbuf1)
    dbufs = (dbuf, dbuf1)
    sem_s = (sem_s0, sem_s1)
    sem_d = (sem_d0, sem_d1)

    def zero_rbuf():
        @pl.loop(0, EB * 2, unroll=4)
        def _(i):
            rbuf[i // 2, pl.ds((i % 2) * L, L)] = jnp.zeros((L,), jnp.float32)

    @pl.loop(0, NCHUNK // L, unroll=4)
    def _(i):
        _zero16(ivd, i * L)

    @pl.loop(0, EB // L, unroll=4)
    def _(i):
        ones[pl.ds(i * L, L)] = jnp.ones((L,), jnp.float32)

    pltpu.sync_copy(ivd, degS.at[pl.ds(nbase, NCHUNK)])
    plsc.subcore_barrier()

    @pl.loop(0, NB)
    def _(i):
        pltpu.sync_copy(dst_hbm.at[pl.ds(ebase + i * EB, EB)], dbuf)
        pltpu.sync_copy(ones, degS.at[dbuf], add=True)

    plsc.subcore_barrier()

    pltpu.sync_copy(degS.at[pl.ds(nbase, NCHUNK)], dch)

    @pl.when(cid == 0)
    def _():
        pltpu.sync_copy(dch, deg_out.at[pl.ds(nbase, NCHUNK)])

    @pl.loop(0, NCHUNK // L, unroll=2)
    def _(i):
        v = dch[pl.ds(i * L, L)]
        bits = plsc.bitcast(v, jnp.int32)
        y = plsc.bitcast(0x5F3759DF - (bits >> 1), jnp.float32)
        y = y * (1.5 - 0.5 * v * y * y)
        y = y * (1.5 - 0.5 * v * y * y)
        y = y * (1.5 - 0.5 * v * y * y)
        ivd[pl.ds(i * L, L)] = 1.0 / v
        dch[pl.ds(i * L, L)] = y

    zero_rbuf()

    @pl.loop(0, NSUB)
    def _(j):
        rb = nbase + j * SUB
        pltpu.sync_copy(h_hbm.at[pl.ds(coff + rb, SUB)], abuf)

        @pl.loop(0, SUB, unroll=2)
        def _(r):
            sc = _splat(dch, j * SUB + r)
            abuf[r, pl.ds(0, L)] = abuf[r, pl.ds(0, L)] * sc
            abuf[r, pl.ds(L, L)] = abuf[r, pl.ds(L, L)] * sc

        pltpu.sync_copy(abuf, pA.at[pl.ds(coff + rb, SUB)])
        pltpu.sync_copy(rbuf.at[pl.ds(0, SUB), :], agg.at[pl.ds(rb, SUB)])

    plsc.subcore_barrier()

    def i_start(i, b):
        e0 = ebase + i * EB
        pltpu.async_copy(src_hbm.at[pl.ds(e0, EB)], sbufs[b], sem_s[b])
        pltpu.async_copy(dst_hbm.at[pl.ds(e0, EB)], dbufs[b], sem_d[b])

    def i_wait(b):
        pltpu.make_async_copy(src_hbm.at[pl.ds(ebase, EB)], sbufs[b],
                              sem_s[b]).wait()
        pltpu.make_async_copy(dst_hbm.at[pl.ds(ebase, EB)], dbufs[b],
                              sem_d[b]).wait()

    def addcoff(b):
        @pl.loop(0, EB // L, unroll=4)
        def _(q):
            sbufs[b][pl.ds(q * L, L)] = sbufs[b][pl.ds(q * L, L)] + coff

    def do_batch(b, tab):
        addcoff(b)
        g = pltpu.async_copy(tab.at[sbufs[b]], rbuf, sem)
        g.wait()
        pltpu.sync_copy(rbuf, agg.at[dbufs[b]], add=True)

    def scatter_pass(tab):
        i_start(0, 0)

        @pl.loop(0, NB // 2)
        def _(kk):
            i0 = 2 * kk
            i_wait(0)
            i_start(i0 + 1, 1)
            do_batch(0, tab)
            i_wait(1)
            i_start(i0 + 2, 0)
            do_batch(1, tab)

        i_wait(0)
        plsc.subcore_barrier()

    def rescale_pass(k, tab_w):
        zero_rbuf()

        @pl.loop(0, NSUB)
        def _(j):
            rb = nbase + j * SUB
            pltpu.sync_copy(agg.at[pl.ds(rb, SUB)], abuf)
            pltpu.sync_copy(rbuf.at[pl.ds(0, SUB), :], agg.at[pl.ds(rb, SUB)])
            pltpu.sync_copy(abuf, s_out.at[k].at[pl.ds(coff + rb, SUB)])

            @pl.loop(0, SUB, unroll=2)
            def _(r):
                sc = _splat(ivd, j * SUB + r)
                abuf[r, pl.ds(0, L)] = abuf[r, pl.ds(0, L)] * sc
                abuf[r, pl.ds(L, L)] = abuf[r, pl.ds(L, L)] * sc

            pltpu.sync_copy(abuf, tab_w.at[pl.ds(coff + rb, SUB)])

        plsc.subcore_barrier()

    @pl.loop(0, K // 2)
    def _(kk):
        scatter_pass(pA)
        rescale_pass(2 * kk, pB)
        scatter_pass(pB)
        rescale_pass(2 * kk + 1, pA)


def _sc_hops(src, dst, h2):
    mesh = plsc.VectorSubcoreMesh(core_axis_name="c", subcore_axis_name="s",
                                  num_cores=NC, num_subcores=NT)
    f = functools.partial(
        pl.kernel,
        out_type=[
            jax.ShapeDtypeStruct((K, NC * NPAD, HALF), jnp.float32),
            jax.ShapeDtypeStruct((NPAD,), jnp.float32),
            jax.ShapeDtypeStruct((NC * NPAD, HALF), jnp.float32),
            jax.ShapeDtypeStruct((NC * NPAD, HALF), jnp.float32),
        ],
        mesh=mesh,
        scratch_types=[
            pltpu.MemorySpace.VMEM_SHARED((NPAD, HALF), jnp.float32),
            pltpu.MemorySpace.VMEM_SHARED((NPAD,), jnp.float32),
            pltpu.VMEM((EB,), jnp.int32),
            pltpu.VMEM((EB,), jnp.int32),
            pltpu.VMEM((EB,), jnp.int32),
            pltpu.VMEM((EB,), jnp.int32),
            pltpu.VMEM((EB, HALF), jnp.float32),
            pltpu.VMEM((SUB, HALF), jnp.float32),
            pltpu.VMEM((NCHUNK,), jnp.float32),
            pltpu.VMEM((NCHUNK,), jnp.float32),
            pltpu.VMEM((EB,), jnp.float32),
            pltpu.SemaphoreType.DMA,
            pltpu.SemaphoreType.DMA,
            pltpu.SemaphoreType.DMA,
            pltpu.SemaphoreType.DMA,
            pltpu.SemaphoreType.DMA,
        ],
        compiler_params=pltpu.CompilerParams(needs_layout_passes=False,
                                             use_tc_tiling_on_sc=False),
    )(_sc_body)
    return f(src, dst, h2)


def _combine_body(deg_ref, h_ref, s_ref, sv_ref, o_ref):
    norm = lax.rsqrt(deg_ref[...])
    sv = sv_ref[...]
    f0 = h_ref[...]
    w0 = jax.nn.sigmoid(jnp.sum(f0 * sv, axis=1, keepdims=True))
    acc = w0 * f0
    for k in range(K):
        fk = jnp.concatenate([s_ref[k, 0], s_ref[k, 1]], axis=1) * norm
        wk = jax.nn.sigmoid(jnp.sum(fk * sv, axis=1, keepdims=True))
        acc = acc + wk * fk
    o_ref[...] = acc


def _combine(deg, h, s_all, svp):
    return pl.pallas_call(
        _combine_body,
        grid=(NPAD // BC,),
        in_specs=[
            pl.BlockSpec((BC, 1), lambda i: (i, 0)),
            pl.BlockSpec((BC, PAD_DIM), lambda i: (i, 0)),
            pl.BlockSpec((K, 2, BC, HALF), lambda i: (0, 0, i, 0)),
            pl.BlockSpec((1, PAD_DIM), lambda i: (0, 0)),
        ],
        out_specs=pl.BlockSpec((BC, PAD_DIM), lambda i: (i, 0)),
        out_shape=jax.ShapeDtypeStruct((NPAD, PAD_DIM), jnp.float32),
        compiler_params=pltpu.CompilerParams(
            dimension_semantics=("parallel",)),
    )(deg, h, s_all, svp)


def kernel(feats, edge_index, W1, b1, W2, b2, W3, b3, s):
    src = jnp.pad(edge_index[0], (0, 2 * EB))
    dst = jnp.pad(edge_index[1], (0, 2 * EB))
    W3p = jnp.pad(W3, ((0, 0), (0, PAD_DIM - OUT_DIM)))
    b3p = jnp.pad(b3, (0, PAD_DIM - OUT_DIM))
    svp = jnp.pad(s[:, 0], (0, PAD_DIM - OUT_DIM))[None, :]

    h = _mlp(feats, W1, b1, W2, b2, W3p, b3p)
    hp = jnp.pad(h, ((0, NPAD - N), (0, 0)))
    h2 = jnp.concatenate([hp[:, :HALF], hp[:, HALF:]], axis=0)

    s_out, deg, _pa, _pb = _sc_hops(src, dst, h2)
    s_all = s_out.reshape(K, 2, NPAD, HALF)

    out = _combine(deg[:, None], hp, s_all, svp)
    return out[:N, :OUT_DIM]
